# baseline (speedup 1.0000x reference)
import functools
import jax
import jax.numpy as jnp
from jax import lax
from jax.experimental import pallas as pl
from jax.experimental.pallas import tpu as pltpu
from jax.experimental.pallas import tpu_sc as plsc

CH = 16
UN = 16


def _make_sc(batch, seq_len, d_model):
    n_workers = 32
    seq_per_w = seq_len // n_workers
    n_chunks = seq_per_w // CH
    assert batch == 4 and n_chunks % 2 == 0
    n_steps = n_chunks * batch
    n_groups = n_steps // 8
    rows = batch * seq_len
    tile = CH * d_model
    mesh = plsc.VectorSubcoreMesh(
        core_axis_name="c", subcore_axis_name="s", num_cores=2, num_subcores=16
    )

    @functools.partial(
        pl.kernel,
        out_type=jax.ShapeDtypeStruct((rows * d_model,), jnp.float32),
        mesh=mesh,
        scratch_types=[
            pltpu.VMEM((tile,), jnp.float32),
            pltpu.VMEM((tile,), jnp.float32),
            pltpu.VMEM((tile,), jnp.float32),
            pltpu.VMEM((tile,), jnp.float32),
            pltpu.VMEM((tile,), jnp.float32),
            pltpu.VMEM((tile,), jnp.float32),
            pltpu.SemaphoreType.DMA((2,)),
            pltpu.SemaphoreType.DMA((4,)),
            pltpu.SemaphoreType.DMA((4,)),
        ],
    )
    def k(x_hbm, pos_hbm, out_hbm, pb0, pb1, xb0, xb1, xb2, xb3,
          psem, lsem, ssem):
        PB = (pb0, pb1)
        XB = (xb0, xb1, xb2, xb3)
        wid = lax.axis_index("s") * 2 + lax.axis_index("c")
        seq0 = wid * seq_per_w

        def x_off(i):
            c = i // batch
            b = i % batch
            return (b * seq_len + seq0 + c * CH) * d_model

        def load_copy(i, s):
            return pltpu.make_async_copy(
                x_hbm.at[pl.ds(x_off(i), tile)], XB[s], lsem.at[s]
            )

        def pload_copy(c, s):
            return pltpu.make_async_copy(
                pos_hbm.at[pl.ds((seq0 + c * CH) * d_model, tile)],
                PB[s],
                psem.at[s],
            )

        def store_copy(i, s):
            return pltpu.make_async_copy(
                XB[s], out_hbm.at[pl.ds(x_off(i), tile)], ssem.at[s]
            )

        def add(pb, xb):
            @plsc.parallel_loop(0, tile // 16, unroll=UN)
            def _(f):
                sl = pl.ds(f * 16, 16)
                xb[sl] = xb[sl] + pb[sl]

        pload_copy(0, 0).start()
        load_copy(0, 0).start()
        load_copy(1, 1).start()

        def group_body(g, _):
            i0 = g * 8
            for kk in range(8):
                i = i0 + kk
                ps = kk // 4
                xs = kk % 4

                if kk == 0:
                    pload_copy(g * 2 + 1, 1).start()
                    pload_copy(g * 2, 0).wait()
                if kk == 4:
                    pload_copy(g * 2 + 1, 1).wait()

                    @pl.when(g + 1 < n_groups)
                    def _():
                        pload_copy(g * 2 + 2, 0).start()

                load_copy(i, xs).wait()
                add(PB[ps], XB[xs])
                store_copy(i, xs).start()

                t = i + 2
                ts = (kk + 2) % 4

                @pl.when(t < n_steps)
                def _():
                    @pl.when(i >= 2)
                    def _():
                        store_copy(i - 2, ts).wait()

                    load_copy(t, ts).start()
            return ()

        lax.fori_loop(0, n_groups, group_body, ())
        store_copy(n_steps - 4, (n_steps - 4) % 4).wait()
        store_copy(n_steps - 3, (n_steps - 3) % 4).wait()
        store_copy(n_steps - 2, (n_steps - 2) % 4).wait()
        store_copy(n_steps - 1, (n_steps - 1) % 4).wait()

    return k


def kernel(x, pos_table):
    batch, seq_len, d_model = x.shape
    x2 = x.reshape(batch * seq_len * d_model)
    pos1 = pos_table.reshape(seq_len * d_model)
    out = _make_sc(batch, seq_len, d_model)(x2, pos1)
    return out.reshape(batch, seq_len, d_model)


# speedup vs baseline: 4.2149x; 4.2149x over previous
import jax
import jax.numpy as jnp
from jax.experimental import pallas as pl


def _add_kernel(x_ref, pos_ref, out_ref):
    out_ref[...] = x_ref[...] + pos_ref[...][None, :, :]


def kernel(x, pos_table):
    batch, seq_len, d_model = x.shape
    seq_blk = 2048
    grid = (seq_len // seq_blk, batch)
    return pl.pallas_call(
        _add_kernel,
        grid=grid,
        in_specs=[
            pl.BlockSpec((1, seq_blk, d_model), lambda s, b: (b, s, 0)),
            pl.BlockSpec((seq_blk, d_model), lambda s, b: (s, 0)),
        ],
        out_specs=pl.BlockSpec((1, seq_blk, d_model), lambda s, b: (b, s, 0)),
        out_shape=jax.ShapeDtypeStruct(x.shape, x.dtype),
    )(x, pos_table)
